# Initial kernel scaffold; baseline (speedup 1.0000x reference)
import jax
import jax.numpy as jnp
from jax.experimental import pallas as pl


def kernel(e_rec, s_rec, r_rec, n_rec, e_lig, s_lig, r_lig, n_lig, e_int, s_int, r_int, params):
    raise NotImplementedError("write your pallas kernel here")


# speedup vs baseline: 5.0140x; 5.0140x over previous
import functools

import jax
import jax.numpy as jnp
from jax import lax
from jax.experimental import pallas as pl
from jax.experimental.pallas import tpu as pltpu
from jax.experimental.pallas import tpu_sc as plsc

C = 128
H = 8
DH = 16
N = 800
NWORK = 32
CH = 64
EB = 256
F32 = jnp.float32


def _sel_dh_to_h():
    d = lax.broadcasted_iota(jnp.int32, (C, 16), 0)
    h = lax.broadcasted_iota(jnp.int32, (C, 16), 1)
    return jnp.where(d // DH == h, 1.0, 0.0).astype(F32)


def _sel_h_to_dh():
    h = lax.broadcasted_iota(jnp.int32, (16, C), 0)
    d = lax.broadcasted_iota(jnp.int32, (16, C), 1)
    return jnp.where((d // DH == h) & (h < H), 1.0, 0.0).astype(F32)


def _headmax16(q, k):
    m16 = jnp.zeros((N, 16), F32)
    for h in range(H):
        qh = q[:, h * DH:(h + 1) * DH]
        kh = k[:, h * DH:(h + 1) * DH]
        at = lax.dot_general(kh, qh, (((1,), (1,)), ((), ())),
                             preferred_element_type=F32)
        mh = jnp.max(at, axis=1, keepdims=True) * 0.25
        hh = lax.broadcasted_iota(jnp.int32, (1, 16), 1)
        oh = jnp.where(hh == h, 1.0, 0.0).astype(F32)
        m16 = m16 + jnp.dot(mh, oh, preferred_element_type=F32)
    return m16


def _nodesm(nodes, wq, wk):
    q = jnp.dot(nodes, wq[...], preferred_element_type=F32)
    k = jnp.dot(nodes, wk[...], preferred_element_type=F32)
    m16 = _headmax16(q, k)
    return jnp.concatenate([nodes, m16, jnp.zeros((N, 112), F32)], axis=1)


def _t1_body(n_rec, n_lig, wn, bn, wq, wk, we, be, w1e_s, b1_s, w1e_i, b1_i,
             nodes_o, nodesm_o, w1fs_o, b1fs_o, w1fi_o, b1fi_o):
    nr = jnp.dot(n_rec[...], wn[...], preferred_element_type=F32) + bn[...]
    nl = jnp.dot(n_lig[...], wn[...], preferred_element_type=F32) + bn[...]
    nodes = jnp.concatenate([nr, nl], axis=0)
    nodes_o[...] = nodes
    nodesm_o[...] = _nodesm(nodes, wq, wk)
    w1fs_o[...] = jnp.dot(we[...], w1e_s[...], preferred_element_type=F32)
    b1fs_o[...] = jnp.dot(be[...], w1e_s[...], preferred_element_type=F32) + b1_s[...]
    w1fi_o[...] = jnp.dot(we[...], w1e_i[...], preferred_element_type=F32)
    b1fi_o[...] = jnp.dot(be[...], w1e_i[...], preferred_element_type=F32) + b1_i[...]


def _t1(n_rec, n_lig, p):
    args = (n_rec, n_lig, p['n_enc']['W'], p['n_enc']['b'][None, :],
            p['single']['Wq'], p['single']['Wk'],
            p['e_enc']['W'], p['e_enc']['b'][None, :],
            p['single']['edge']['W1'][:C], p['single']['edge']['b1'][None, :],
            p['inter']['edge']['W1'][:C], p['inter']['edge']['b1'][None, :])
    outs = [jax.ShapeDtypeStruct((N, C), F32), jax.ShapeDtypeStruct((N, 256), F32),
            jax.ShapeDtypeStruct((16, C), F32), jax.ShapeDtypeStruct((1, C), F32),
            jax.ShapeDtypeStruct((16, C), F32), jax.ShapeDtypeStruct((1, C), F32)]
    return pl.pallas_call(_t1_body, out_shape=outs)(*args)


def _nodeup_body(nodes, mw0, wna, wnb, bn1, wn2, bn2):
    mwp = mw0[...]
    aggu = mwp[:, :C]
    den16 = mwp[:, C:C + 16]
    r16 = _sel_h_to_dh()
    denr = jnp.dot(den16, r16, preferred_element_type=F32) + 1e-9
    agg = aggu / denr
    h1 = jnp.maximum(
        jnp.dot(nodes[...], wna[...], preferred_element_type=F32)
        + jnp.dot(agg, wnb[...], preferred_element_type=F32) + bn1[...], 0.0)
    return jnp.dot(h1, wn2[...], preferred_element_type=F32) + bn2[...]


def _node_mlp_args(nodes, mwp, pst):
    w1 = pst['node']['W1']
    return (nodes, mwp,
            w1[:C], w1[C:], pst['node']['b1'][None, :],
            pst['node']['W2'], pst['node']['b2'][None, :])


def _t2_body(nodes, mw0, wna, wnb, bn1, wn2, bn2,
             colsums, wg_n, wg_e, bg, wq, wk,
             ns_o, nodesm_o, gr_o, gl_o):
    ns = _nodeup_body(nodes, mw0, wna, wnb, bn1, wn2, bn2)
    ns_o[...] = ns
    nodesm_o[...] = _nodesm(ns, wq, wk)
    nrm = jnp.mean(ns[0:400], axis=0, keepdims=True)
    nlm = jnp.mean(ns[400:800], axis=0, keepdims=True)
    erm = jnp.sum(colsums[0:200], axis=0) * (1.0 / 51200.0)
    elm = jnp.sum(colsums[200:400], axis=0) * (1.0 / 51200.0)
    gr_o[...] = (jnp.dot(nrm, wg_n[...], preferred_element_type=F32)
                 + jnp.dot(erm, wg_e[...], preferred_element_type=F32) + bg[...])
    gl_o[...] = (jnp.dot(nlm, wg_n[...], preferred_element_type=F32)
                 + jnp.dot(elm, wg_e[...], preferred_element_type=F32) + bg[...])


def _t2(nodes, mwp, colsums, p):
    pst = p['single']
    wg = pst['glob']['W']
    args = (*_node_mlp_args(nodes, mwp, pst), colsums,
            wg[:C], wg[C:], pst['glob']['b'][None, :],
            p['inter']['Wq'], p['inter']['Wk'])
    outs = [jax.ShapeDtypeStruct((N, C), F32), jax.ShapeDtypeStruct((N, 256), F32),
            jax.ShapeDtypeStruct((1, C), F32), jax.ShapeDtypeStruct((1, C), F32)]
    return pl.pallas_call(_t2_body, out_shape=outs)(*args)


def _t3_body(nodes, mw0, wna, wnb, bn1, wn2, bn2,
             colsums, wg_n, wg_e, bg, wq, wk,
             nodesd_o, nodesm_o, gi_o):
    ni = _nodeup_body(nodes, mw0, wna, wnb, bn1, wn2, bn2)
    nodesd = nodes[...] + ni
    nodesd_o[...] = nodesd
    nodesm_o[...] = _nodesm(nodesd, wq, wk)
    nim = jnp.mean(ni, axis=0, keepdims=True)
    eim = jnp.sum(colsums[...], axis=0) * (1.0 / 51200.0)
    gi_o[...] = (jnp.dot(nim, wg_n[...], preferred_element_type=F32)
                 + jnp.dot(eim, wg_e[...], preferred_element_type=F32) + bg[...])


def _t3(nodes, mwp, colsums, p):
    pst = p['inter']
    wg = pst['glob']['W']
    args = (*_node_mlp_args(nodes, mwp, pst), colsums,
            wg[:C], wg[C:], pst['glob']['b'][None, :],
            p['dock']['Wq'], p['dock']['Wk'])
    outs = [jax.ShapeDtypeStruct((N, C), F32), jax.ShapeDtypeStruct((N, 256), F32),
            jax.ShapeDtypeStruct((1, C), F32)]
    return pl.pallas_call(_t3_body, out_shape=outs)(*args)


def _sigmoid(x):
    return 1.0 / (1.0 + jnp.exp(-x))


def _mlp_in(x, pm):
    return (pm['W1'], pm['b1'][None, :], pm['W2'], pm['b2'][None, :])


def _t4_body(nodes, mw0, mw1,
             wna, wnb, bn1, wn2, bn2, colsa, colsb, wg_n, wg_e, bg,
             rw1, rb1, rw2, rb2, tw1, tb1, tw2, tb2, cw1, cb1, cw2, cb2,
             wtr, btr, wro, bro, wco, bco, out_o):
    mwp = mw0[...] + mw1[...]
    aggu = mwp[:, :C]
    den16 = mwp[:, C:C + 16]
    r16 = _sel_h_to_dh()
    denr = jnp.dot(den16, r16, preferred_element_type=F32) + 1e-9
    agg = aggu / denr
    h1 = jnp.maximum(
        jnp.dot(nodes[...], wna[...], preferred_element_type=F32)
        + jnp.dot(agg, wnb[...], preferred_element_type=F32) + bn1[...], 0.0)
    nd = jnp.dot(h1, wn2[...], preferred_element_type=F32) + bn2[...]
    ndm = jnp.mean(nd, axis=0, keepdims=True)
    edm = ((jnp.sum(colsa[...], axis=0) + jnp.sum(colsb[...], axis=0))
           * (1.0 / 153600.0))
    gd = (jnp.dot(ndm, wg_n[...], preferred_element_type=F32)
          + jnp.dot(edm, wg_e[...], preferred_element_type=F32) + bg[...])

    def mlp(x, w1, b1, w2, b2):
        hh = jnp.maximum(jnp.dot(x, w1[...], preferred_element_type=F32) + b1[...], 0.0)
        return jnp.dot(hh, w2[...], preferred_element_type=F32) + b2[...]

    out_r = mlp(gd, rw1, rb1, rw2, rb2)
    out_t = mlp(gd + out_r, tw1, tb1, tw2, tb2)
    out_c = mlp(gd, cw1, cb1, cw2, cb2)
    t3 = _sigmoid(jnp.dot(out_t, wtr[...], preferred_element_type=F32) + btr[...]) * 2.0 - 1.0
    r3 = _sigmoid(jnp.dot(out_r, wro[...], preferred_element_type=F32) + bro[...]) * 0.2 - 0.1
    c1 = _sigmoid(jnp.dot(out_c, wco[...], preferred_element_type=F32) + bco[...])
    i = lax.broadcasted_iota(jnp.int32, (16, 16), 0)
    j = lax.broadcasted_iota(jnp.int32, (16, 16), 1)
    p_r = jnp.where((i == j) & (i < 3), 1.0, 0.0).astype(F32)
    p_t = jnp.where((j == i + 3) & (i < 3), 1.0, 0.0).astype(F32)
    p_c = jnp.where((i == 0) & (j == 6), 1.0, 0.0).astype(F32)
    out_o[...] = (jnp.dot(r3, p_r, preferred_element_type=F32)
                  + jnp.dot(t3, p_t, preferred_element_type=F32)
                  + jnp.dot(c1, p_c, preferred_element_type=F32))


def _pad16(w):
    return jnp.pad(w, ((0, 0), (0, 16 - w.shape[1])))


def _t4(nodes, mwps, colsa, colsb, p):
    pst = p['dock']
    w1 = pst['node']['W1']
    wg = pst['glob']['W']
    args = (nodes, mwps[0], mwps[1],
            w1[:C], w1[C:], pst['node']['b1'][None, :],
            pst['node']['W2'], pst['node']['b2'][None, :],
            colsa, colsb, wg[:C], wg[C:], pst['glob']['b'][None, :],
            *_mlp_in(None, p['out_r']), *_mlp_in(None, p['out_t']),
            *_mlp_in(None, p['out_c']),
            _pad16(p['traslator']['W']), _pad16(p['traslator']['b'][None, :]),
            _pad16(p['rotator']['W']), _pad16(p['rotator']['b'][None, :]),
            _pad16(p['confidence']['W']), _pad16(p['confidence']['b'][None, :]))
    out = pl.pallas_call(_t4_body, out_shape=[jax.ShapeDtypeStruct((1, 16), F32)])(*args)
    return out[0]


def _p4_body_mk(din, write_enew):
    def body(*refs):
        if write_enew:
            (eraw, gs, grm, rcol, w1f, b1f, w1s, w1r, wq, wk, w2, b2,
             enew_o, agg_o, cols_o) = refs
        else:
            (eraw, gs, grm, rcol, w1f, b1f, w1s, w1r, wq, wk, w2, b2,
             agg_o, cols_o) = refs
        x = jnp.dot(eraw[...], w1f[...], preferred_element_type=F32) + b1f[...]
        gsv = gs[...]
        grmv = grm[...]
        grv = grmv[:, :C]
        gm = grmv[:, C:C + 16]
        h1 = jnp.maximum(x + jnp.dot(gsv, w1s[...], preferred_element_type=F32)
                         + jnp.dot(grv, w1r[...], preferred_element_type=F32), 0.0)
        enew = jnp.dot(h1, w2[...], preferred_element_type=F32) + b2[...]
        q = jnp.dot(gsv, wq[...], preferred_element_type=F32)
        k = jnp.dot(grv, wk[...], preferred_element_type=F32)
        l16 = jnp.dot(q * k, _sel_dh_to_h(), preferred_element_type=F32) * 0.25
        w16 = jnp.exp(l16 - gm)
        msgu = enew * jnp.dot(w16, _sel_h_to_dh(), preferred_element_type=F32)
        nn = lax.broadcasted_iota(jnp.int32, (EB, N), 1)
        onehot = jnp.where(rcol[...] == nn, 1.0, 0.0).astype(jnp.bfloat16)
        mw = jnp.concatenate([msgu, w16[:, :16]], axis=1).astype(jnp.bfloat16)
        contrib = lax.dot_general(onehot, mw, (((0,), (0,)), ((), ())),
                                  preferred_element_type=F32)

        @pl.when(pl.program_id(0) == 0)
        def _zero():
            agg_o[...] = jnp.zeros((N, 144), F32)

        agg_o[...] += contrib
        if write_enew:
            enew_o[...] = enew
        cols_o[...] = jnp.sum(enew, axis=0, keepdims=True)[None, :, :]
    return body


def _p4(eraw, gs, grm, rcol, w1f, b1f, pst, write_enew):
    e = eraw.shape[0]
    din = eraw.shape[1]
    nb = e // EB
    w1 = pst['edge']['W1']
    grid = (nb,)
    full = lambda s: pl.BlockSpec(s, lambda i: (0, 0))
    in_specs = [pl.BlockSpec((EB, din), lambda i: (i, 0)),
                pl.BlockSpec((EB, C), lambda i: (i, 0)),
                pl.BlockSpec((EB, 256), lambda i: (i, 0)),
                pl.BlockSpec((EB, 1), lambda i: (i, 0)),
                full((din, C)), full((1, C)), full((C, C)), full((C, C)),
                full((C, C)), full((C, C)), full((C, C)), full((1, C))]
    out_shapes = []
    out_specs = []
    if write_enew:
        out_shapes.append(jax.ShapeDtypeStruct((e, C), F32))
        out_specs.append(pl.BlockSpec((EB, C), lambda i: (i, 0)))
    out_shapes += [jax.ShapeDtypeStruct((N, 144), F32),
                   jax.ShapeDtypeStruct((nb, 1, C), F32)]
    out_specs += [pl.BlockSpec((N, 144), lambda i: (0, 0)),
                  pl.BlockSpec((1, 1, C), lambda i: (i, 0, 0))]
    args = (eraw, gs, grm, rcol, w1f, b1f, w1[C:2 * C], w1[2 * C:],
            pst['Wq'], pst['Wk'], pst['edge']['W2'], pst['edge']['b2'][None, :])
    return pl.pallas_call(
        _p4_body_mk(din, write_enew), grid=grid,
        in_specs=in_specs, out_specs=out_specs, out_shape=out_shapes)(*args)


def _mesh():
    return plsc.VectorSubcoreMesh(core_axis_name="c", subcore_axis_name="s")


def _sc_gather(table, tablem, sidx, ridx):
    nch = sidx.shape[1]
    e = NWORK * nch * CH
    per = nch * CH

    @functools.partial(
        pl.kernel, mesh=_mesh(),
        out_type=[jax.ShapeDtypeStruct((e, C), F32),
                  jax.ShapeDtypeStruct((e, 256), F32)],
        scratch_types=[pltpu.VMEM((nch, CH), jnp.int32),
                       pltpu.VMEM((nch, CH), jnp.int32),
                       pltpu.VMEM((CH, C), F32),
                       pltpu.VMEM((CH, 256), F32),
                       pltpu.SemaphoreType.DMA,
                       pltpu.SemaphoreType.DMA],
    )
    def k(table_h, tablem_h, sidx_h, ridx_h, gs_h, grm_h,
          sv, rv, bs, br, sem1, sem2):
        wid = lax.axis_index("s") * 2 + lax.axis_index("c")
        base = wid * per
        pltpu.sync_copy(sidx_h.at[wid], sv)
        pltpu.sync_copy(ridx_h.at[wid], rv)

        def body(kk, carry):
            c1 = pltpu.async_copy(table_h.at[sv.at[kk]], bs, sem1)
            c2 = pltpu.async_copy(tablem_h.at[rv.at[kk]], br, sem2)
            c1.wait()
            c2.wait()
            pltpu.sync_copy(bs, gs_h.at[pl.ds(base + kk * CH, CH)])
            pltpu.sync_copy(br, grm_h.at[pl.ds(base + kk * CH, CH)])
            return carry

        lax.fori_loop(0, nch, body, 0)

    return k(table, tablem, sidx, ridx)


def _tiles(idx):
    return idx.astype(jnp.int32).reshape(NWORK, -1, CH)


def _stripe_perm(r):
    e = r.shape[0]
    nch = e // CH
    perm = jnp.argsort(r)
    return perm.reshape(CH, nch).T.reshape(-1)


def kernel(e_rec, s_rec, r_rec, n_rec, e_lig, s_lig, r_lig, n_lig,
           e_int, s_int, r_int, params):
    p = params

    s_s = jnp.concatenate([s_rec, s_lig + 400])
    r_s = jnp.concatenate([r_rec, r_lig + 400])
    sidx_s, ridx_s = _tiles(s_s), _tiles(r_s)
    sidx_i, ridx_i = _tiles(s_int), _tiles(r_int)
    rcol_s = r_s.astype(jnp.int32)[:, None]
    rcol_i = r_int.astype(jnp.int32)[:, None]

    nodes_s, nodesm_s, w1fs, b1fs, w1fi, b1fi = _t1(n_rec, n_lig, p)
    gs, grm = _sc_gather(nodes_s, nodesm_s, sidx_s, ridx_s)
    eraw_s = jnp.concatenate([e_rec, e_lig], axis=0)
    enew_s, mwp, cols_s = _p4(eraw_s, gs, grm, rcol_s, w1fs, b1fs,
                              p['single'], True)
    ns, nodesm_i, gr_g, gl_g = _t2(nodes_s, mwp, cols_s, p)

    gs, grm = _sc_gather(ns, nodesm_i, sidx_i, ridx_i)
    enew_i, mwp, cols_i = _p4(e_int, gs, grm, rcol_i, w1fi, b1fi,
                              p['inter'], True)
    nodes_d, nodesm_d, gi_g = _t3(ns, mwp, cols_i, p)

    w1d = p['dock']['edge']['W1'][:C]
    b1d = p['dock']['edge']['b1'][None, :]
    gsa, grma = _sc_gather(nodes_d, nodesm_d, sidx_s, ridx_s)
    mwp_a, cols_a = _p4(enew_s, gsa, grma, rcol_s, w1d, b1d, p['dock'], False)
    gsb, grmb = _sc_gather(nodes_d, nodesm_d, sidx_i, ridx_i)
    mwp_b, cols_b = _p4(enew_i, gsb, grmb, rcol_i, w1d, b1d, p['dock'], False)

    out16 = _t4(nodes_d, (mwp_a, mwp_b), cols_a, cols_b, p)
    return out16[0, :7]


# speedup vs baseline: 5.5018x; 1.0973x over previous
import functools

import jax
import jax.numpy as jnp
from jax import lax
from jax.experimental import pallas as pl
from jax.experimental.pallas import tpu as pltpu
from jax.experimental.pallas import tpu_sc as plsc

C = 128
H = 8
DH = 16
N = 800
NWORK = 32
EB = 256
F32 = jnp.float32
BF16 = jnp.bfloat16


def _sel_dh_to_h():
    d = lax.broadcasted_iota(jnp.int32, (C, 16), 0)
    h = lax.broadcasted_iota(jnp.int32, (C, 16), 1)
    return jnp.where(d // DH == h, 1.0, 0.0).astype(F32)


def _sel_h_to_dh():
    h = lax.broadcasted_iota(jnp.int32, (16, C), 0)
    d = lax.broadcasted_iota(jnp.int32, (16, C), 1)
    return jnp.where((d // DH == h) & (h < H), 1.0, 0.0).astype(F32)


def _headmax_global(nodes, wq, wk):
    q = jnp.dot(nodes, wq[...], preferred_element_type=F32)
    k = jnp.dot(nodes, wk[...], preferred_element_type=F32)
    m16 = jnp.zeros((1, 16), F32)
    for h in range(H):
        qh = q[:, h * DH:(h + 1) * DH]
        kh = k[:, h * DH:(h + 1) * DH]
        at = lax.dot_general(kh, qh, (((1,), (1,)), ((), ())),
                             preferred_element_type=F32)
        mh = jnp.max(jnp.max(at, axis=1, keepdims=True), axis=0,
                     keepdims=True) * 0.25
        hh = lax.broadcasted_iota(jnp.int32, (1, 16), 1)
        oh = jnp.where(hh == h, 1.0, 0.0).astype(F32)
        m16 = m16 + mh * oh
    return m16


def _t1_body(n_rec, n_lig, wn, bn, wq, wk, we, be, w1e_s, b1_s, w1e_i, b1_i,
             nodes_o, m16_o, w1fs_o, b1fs_o, w1fi_o, b1fi_o):
    nr = jnp.dot(n_rec[...], wn[...], preferred_element_type=F32) + bn[...]
    nl = jnp.dot(n_lig[...], wn[...], preferred_element_type=F32) + bn[...]
    nodes = jnp.concatenate([nr, nl], axis=0)
    nodes_o[...] = nodes
    m16_o[...] = _headmax_global(nodes, wq, wk)
    w1fs_o[...] = jnp.dot(we[...], w1e_s[...], preferred_element_type=F32)
    b1fs_o[...] = jnp.dot(be[...], w1e_s[...], preferred_element_type=F32) + b1_s[...]
    w1fi_o[...] = jnp.dot(we[...], w1e_i[...], preferred_element_type=F32)
    b1fi_o[...] = jnp.dot(be[...], w1e_i[...], preferred_element_type=F32) + b1_i[...]


def _t1(n_rec, n_lig, p):
    args = (n_rec, n_lig, p['n_enc']['W'], p['n_enc']['b'][None, :],
            p['single']['Wq'], p['single']['Wk'],
            p['e_enc']['W'], p['e_enc']['b'][None, :],
            p['single']['edge']['W1'][:C], p['single']['edge']['b1'][None, :],
            p['inter']['edge']['W1'][:C], p['inter']['edge']['b1'][None, :])
    outs = [jax.ShapeDtypeStruct((N, C), F32), jax.ShapeDtypeStruct((1, 16), F32),
            jax.ShapeDtypeStruct((16, C), F32), jax.ShapeDtypeStruct((1, C), F32),
            jax.ShapeDtypeStruct((16, C), F32), jax.ShapeDtypeStruct((1, C), F32)]
    return pl.pallas_call(_t1_body, out_shape=outs)(*args)


def _nodeup_body(nodes, mw0, wna, wnb, bn1, wn2, bn2):
    mwp = mw0[...]
    aggu = mwp[:, :C]
    den16 = mwp[:, C:C + 16]
    r16 = _sel_h_to_dh()
    denr = jnp.dot(den16, r16, preferred_element_type=F32) + 1e-9
    agg = aggu / denr
    h1 = jnp.maximum(
        jnp.dot(nodes[...], wna[...], preferred_element_type=F32)
        + jnp.dot(agg, wnb[...], preferred_element_type=F32) + bn1[...], 0.0)
    return jnp.dot(h1, wn2[...], preferred_element_type=F32) + bn2[...]


def _node_mlp_args(nodes, mwp, pst):
    w1 = pst['node']['W1']
    return (nodes, mwp,
            w1[:C], w1[C:], pst['node']['b1'][None, :],
            pst['node']['W2'], pst['node']['b2'][None, :])


def _t2_body(nodes, mw0, wna, wnb, bn1, wn2, bn2,
             colsums, wg_n, wg_e, bg, wq, wk,
             ns_o, m16_o, gr_o, gl_o):
    ns = _nodeup_body(nodes, mw0, wna, wnb, bn1, wn2, bn2)
    ns_o[...] = ns
    m16_o[...] = _headmax_global(ns, wq, wk)
    nrm = jnp.mean(ns[0:400], axis=0, keepdims=True)
    nlm = jnp.mean(ns[400:800], axis=0, keepdims=True)
    erm = jnp.sum(colsums[0:200], axis=0) * (1.0 / 51200.0)
    elm = jnp.sum(colsums[200:400], axis=0) * (1.0 / 51200.0)
    gr_o[...] = (jnp.dot(nrm, wg_n[...], preferred_element_type=F32)
                 + jnp.dot(erm, wg_e[...], preferred_element_type=F32) + bg[...])
    gl_o[...] = (jnp.dot(nlm, wg_n[...], preferred_element_type=F32)
                 + jnp.dot(elm, wg_e[...], preferred_element_type=F32) + bg[...])


def _t2(nodes, mwp, colsums, p):
    pst = p['single']
    wg = pst['glob']['W']
    args = (*_node_mlp_args(nodes, mwp, pst), colsums,
            wg[:C], wg[C:], pst['glob']['b'][None, :],
            p['inter']['Wq'], p['inter']['Wk'])
    outs = [jax.ShapeDtypeStruct((N, C), F32), jax.ShapeDtypeStruct((1, 16), F32),
            jax.ShapeDtypeStruct((1, C), F32), jax.ShapeDtypeStruct((1, C), F32)]
    return pl.pallas_call(_t2_body, out_shape=outs)(*args)


def _t3_body(nodes, mw0, wna, wnb, bn1, wn2, bn2,
             colsums, wg_n, wg_e, bg, wq, wk,
             nodesd_o, m16_o, gi_o):
    ni = _nodeup_body(nodes, mw0, wna, wnb, bn1, wn2, bn2)
    nodesd = nodes[...] + ni
    nodesd_o[...] = nodesd
    m16_o[...] = _headmax_global(nodesd, wq, wk)
    nim = jnp.mean(ni, axis=0, keepdims=True)
    eim = jnp.sum(colsums[...], axis=0) * (1.0 / 51200.0)
    gi_o[...] = (jnp.dot(nim, wg_n[...], preferred_element_type=F32)
                 + jnp.dot(eim, wg_e[...], preferred_element_type=F32) + bg[...])


def _t3(nodes, mwp, colsums, p):
    pst = p['inter']
    wg = pst['glob']['W']
    args = (*_node_mlp_args(nodes, mwp, pst), colsums,
            wg[:C], wg[C:], pst['glob']['b'][None, :],
            p['dock']['Wq'], p['dock']['Wk'])
    outs = [jax.ShapeDtypeStruct((N, C), F32), jax.ShapeDtypeStruct((1, 16), F32),
            jax.ShapeDtypeStruct((1, C), F32)]
    return pl.pallas_call(_t3_body, out_shape=outs)(*args)


def _sigmoid(x):
    return 1.0 / (1.0 + jnp.exp(-x))


def _mlp_in(pm):
    return (pm['W1'], pm['b1'][None, :], pm['W2'], pm['b2'][None, :])


def _t4_body(nodes, mw0, mw1,
             wna, wnb, bn1, wn2, bn2, colsa, colsb, wg_n, wg_e, bg,
             rw1, rb1, rw2, rb2, tw1, tb1, tw2, tb2, cw1, cb1, cw2, cb2,
             wtr, btr, wro, bro, wco, bco, out_o):
    mwp = mw0[...] + mw1[...]
    aggu = mwp[:, :C]
    den16 = mwp[:, C:C + 16]
    r16 = _sel_h_to_dh()
    denr = jnp.dot(den16, r16, preferred_element_type=F32) + 1e-9
    agg = aggu / denr
    h1 = jnp.maximum(
        jnp.dot(nodes[...], wna[...], preferred_element_type=F32)
        + jnp.dot(agg, wnb[...], preferred_element_type=F32) + bn1[...], 0.0)
    nd = jnp.dot(h1, wn2[...], preferred_element_type=F32) + bn2[...]
    ndm = jnp.mean(nd, axis=0, keepdims=True)
    edm = ((jnp.sum(colsa[...], axis=0) + jnp.sum(colsb[...], axis=0))
           * (1.0 / 153600.0))
    gd = (jnp.dot(ndm, wg_n[...], preferred_element_type=F32)
          + jnp.dot(edm, wg_e[...], preferred_element_type=F32) + bg[...])

    def mlp(x, w1, b1, w2, b2):
        hh = jnp.maximum(jnp.dot(x, w1[...], preferred_element_type=F32) + b1[...], 0.0)
        return jnp.dot(hh, w2[...], preferred_element_type=F32) + b2[...]

    out_r = mlp(gd, rw1, rb1, rw2, rb2)
    out_t = mlp(gd + out_r, tw1, tb1, tw2, tb2)
    out_c = mlp(gd, cw1, cb1, cw2, cb2)
    t3 = _sigmoid(jnp.dot(out_t, wtr[...], preferred_element_type=F32) + btr[...]) * 2.0 - 1.0
    r3 = _sigmoid(jnp.dot(out_r, wro[...], preferred_element_type=F32) + bro[...]) * 0.2 - 0.1
    c1 = _sigmoid(jnp.dot(out_c, wco[...], preferred_element_type=F32) + bco[...])
    i = lax.broadcasted_iota(jnp.int32, (16, 16), 0)
    j = lax.broadcasted_iota(jnp.int32, (16, 16), 1)
    p_r = jnp.where((i == j) & (i < 3), 1.0, 0.0).astype(F32)
    p_t = jnp.where((j == i + 3) & (i < 3), 1.0, 0.0).astype(F32)
    p_c = jnp.where((i == 0) & (j == 6), 1.0, 0.0).astype(F32)
    out_o[...] = (jnp.dot(r3, p_r, preferred_element_type=F32)
                  + jnp.dot(t3, p_t, preferred_element_type=F32)
                  + jnp.dot(c1, p_c, preferred_element_type=F32))


def _pad16(w):
    return jnp.pad(w, ((0, 0), (0, 16 - w.shape[1])))


def _t4(nodes, mwps, colsa, colsb, p):
    pst = p['dock']
    w1 = pst['node']['W1']
    wg = pst['glob']['W']
    args = (nodes, mwps[0], mwps[1],
            w1[:C], w1[C:], pst['node']['b1'][None, :],
            pst['node']['W2'], pst['node']['b2'][None, :],
            colsa, colsb, wg[:C], wg[C:], pst['glob']['b'][None, :],
            *_mlp_in(p['out_r']), *_mlp_in(p['out_t']), *_mlp_in(p['out_c']),
            _pad16(p['traslator']['W']), _pad16(p['traslator']['b'][None, :]),
            _pad16(p['rotator']['W']), _pad16(p['rotator']['b'][None, :]),
            _pad16(p['confidence']['W']), _pad16(p['confidence']['b'][None, :]))
    out = pl.pallas_call(_t4_body, out_shape=[jax.ShapeDtypeStruct((1, 16), F32)])(*args)
    return out[0]


def _p4_body_mk(din, write_enew):
    def body(*refs):
        if write_enew:
            (eraw, gs, gr, rcol, m16, w1f, b1f, wsq, wrk, w2, b2,
             enew_o, agg_o, cols_o) = refs
        else:
            (eraw, gs, gr, rcol, m16, w1f, b1f, wsq, wrk, w2, b2,
             agg_o, cols_o) = refs
        x = jnp.dot(eraw[...], w1f[...], preferred_element_type=F32) + b1f[...]
        gsv = gs[...]
        grv = gr[...]
        ts = jnp.dot(gsv, wsq[...], preferred_element_type=F32)
        tr = jnp.dot(grv, wrk[...], preferred_element_type=F32)
        h1 = jnp.maximum(x + ts[:, :C] + tr[:, :C], 0.0)
        enew = jnp.dot(h1, w2[...], preferred_element_type=F32) + b2[...]
        l16 = jnp.dot(ts[:, C:] * tr[:, C:], _sel_dh_to_h(),
                      preferred_element_type=F32) * 0.25
        w16 = jnp.exp(l16 - m16[...])
        msgu = enew * jnp.dot(w16, _sel_h_to_dh(), preferred_element_type=F32)
        nn = lax.broadcasted_iota(jnp.int32, (EB, N), 1)
        onehot = jnp.where(rcol[...] == nn, 1.0, 0.0).astype(BF16)
        mw = jnp.concatenate([msgu, w16[:, :16]], axis=1).astype(BF16)
        contrib = lax.dot_general(onehot, mw, (((0,), (0,)), ((), ())),
                                  preferred_element_type=F32)

        @pl.when(pl.program_id(0) == 0)
        def _zero():
            agg_o[...] = jnp.zeros((N, 144), F32)

        agg_o[...] += contrib
        if write_enew:
            enew_o[...] = enew
        cols_o[...] = jnp.sum(enew, axis=0, keepdims=True)[None, :, :]
    return body


def _p4(eraw, gs, gr, rcol, m16, w1f, b1f, pst, write_enew):
    e = eraw.shape[0]
    din = eraw.shape[1]
    nb = e // EB
    w1 = pst['edge']['W1']
    wsq = jnp.concatenate([w1[C:2 * C], pst['Wq']], axis=1)
    wrk = jnp.concatenate([w1[2 * C:], pst['Wk']], axis=1)
    full = lambda s: pl.BlockSpec(s, lambda i: tuple(0 for _ in s))
    in_specs = [pl.BlockSpec((EB, din), lambda i: (i, 0)),
                pl.BlockSpec((EB, C), lambda i: (i, 0)),
                pl.BlockSpec((EB, C), lambda i: (i, 0)),
                pl.BlockSpec((EB, 1), lambda i: (i, 0)),
                full((1, 16)),
                full((din, C)), full((1, C)), full((C, 256)), full((C, 256)),
                full((C, C)), full((1, C))]
    out_shapes = []
    out_specs = []
    if write_enew:
        out_shapes.append(jax.ShapeDtypeStruct((e, C), F32))
        out_specs.append(pl.BlockSpec((EB, C), lambda i: (i, 0)))
    out_shapes += [jax.ShapeDtypeStruct((N, 144), F32),
                   jax.ShapeDtypeStruct((nb, 1, C), F32)]
    out_specs += [pl.BlockSpec((N, 144), lambda i: (0, 0)),
                  pl.BlockSpec((1, 1, C), lambda i: (i, 0, 0))]
    args = (eraw, gs, gr, rcol, m16, w1f, b1f, wsq, wrk,
            pst['edge']['W2'], pst['edge']['b2'][None, :])
    return pl.pallas_call(
        _p4_body_mk(din, write_enew), grid=(nb,),
        in_specs=in_specs, out_specs=out_specs, out_shape=out_shapes)(*args)


def _mesh():
    return plsc.VectorSubcoreMesh(core_axis_name="c", subcore_axis_name="s")


def _sc_gather(table, sidx, ridx):
    nch = sidx.shape[1]
    ch = sidx.shape[2]
    e = NWORK * nch * ch
    per = nch * ch

    @functools.partial(
        pl.kernel, mesh=_mesh(),
        out_type=[jax.ShapeDtypeStruct((e, C), F32),
                  jax.ShapeDtypeStruct((e, C), F32)],
        scratch_types=[pltpu.VMEM((nch, ch), jnp.int32),
                       pltpu.VMEM((nch, ch), jnp.int32),
                       pltpu.VMEM((ch, C), F32),
                       pltpu.VMEM((ch, C), F32),
                       pltpu.VMEM((ch, C), F32),
                       pltpu.VMEM((ch, C), F32),
                       pltpu.SemaphoreType.DMA,
                       pltpu.SemaphoreType.DMA,
                       pltpu.SemaphoreType.DMA,
                       pltpu.SemaphoreType.DMA],
    )
    def k(table_h, sidx_h, ridx_h, gs_h, gr_h,
          sv, rv, bs0, br0, bs1, br1, ss0, sr0, ss1, sr1):
        wid = lax.axis_index("s") * 2 + lax.axis_index("c")
        base = wid * per
        pltpu.sync_copy(sidx_h.at[wid], sv)
        pltpu.sync_copy(ridx_h.at[wid], rv)

        def issue(kk, bs, br, sems, semr):
            pltpu.async_copy(table_h.at[sv.at[kk]], bs, sems)
            pltpu.async_copy(table_h.at[rv.at[kk]], br, semr)

        def drain(bs, br, sems, semr):
            pltpu.make_async_copy(table_h.at[pl.ds(0, ch)], bs, sems).wait()
            pltpu.make_async_copy(table_h.at[pl.ds(0, ch)], br, semr).wait()

        def flush(kk, bs, br):
            pltpu.sync_copy(bs, gs_h.at[pl.ds(base + kk * ch, ch)])
            pltpu.sync_copy(br, gr_h.at[pl.ds(base + kk * ch, ch)])

        issue(0, bs0, br0, ss0, sr0)

        def pair(i, carry):
            k0 = i * 2
            issue(k0 + 1, bs1, br1, ss1, sr1)
            drain(bs0, br0, ss0, sr0)
            flush(k0, bs0, br0)

            @pl.when(k0 + 2 < nch)
            def _next():
                issue(k0 + 2, bs0, br0, ss0, sr0)

            drain(bs1, br1, ss1, sr1)
            flush(k0 + 1, bs1, br1)
            return carry

        lax.fori_loop(0, nch // 2, pair, 0)

    return k(table, sidx, ridx)


def _tiles(idx, ch):
    return idx.astype(jnp.int32).reshape(NWORK, -1, ch)


def kernel(e_rec, s_rec, r_rec, n_rec, e_lig, s_lig, r_lig, n_lig,
           e_int, s_int, r_int, params):
    p = params

    s_s = jnp.concatenate([s_rec, s_lig + 400])
    r_s = jnp.concatenate([r_rec, r_lig + 400])
    sidx_s, ridx_s = _tiles(s_s, 64), _tiles(r_s, 64)
    sidx_i, ridx_i = _tiles(s_int, 32), _tiles(r_int, 32)
    rcol_s = r_s.astype(jnp.int32)[:, None]
    rcol_i = r_int.astype(jnp.int32)[:, None]

    nodes_s, m16_s, w1fs, b1fs, w1fi, b1fi = _t1(n_rec, n_lig, p)
    gs, gr = _sc_gather(nodes_s, sidx_s, ridx_s)
    eraw_s = jnp.concatenate([e_rec, e_lig], axis=0)
    enew_s, mwp, cols_s = _p4(eraw_s, gs, gr, rcol_s, m16_s, w1fs, b1fs,
                              p['single'], True)
    ns, m16_i, gr_g, gl_g = _t2(nodes_s, mwp, cols_s, p)

    gs, gr = _sc_gather(ns, sidx_i, ridx_i)
    enew_i, mwp, cols_i = _p4(e_int, gs, gr, rcol_i, m16_i, w1fi, b1fi,
                              p['inter'], True)
    nodes_d, m16_d, gi_g = _t3(ns, mwp, cols_i, p)

    w1d = p['dock']['edge']['W1'][:C]
    b1d = p['dock']['edge']['b1'][None, :]
    gsa, gra = _sc_gather(nodes_d, sidx_s, ridx_s)
    gsb, grb = _sc_gather(nodes_d, sidx_i, ridx_i)
    mwp_a, cols_a = _p4(enew_s, gsa, gra, rcol_s, m16_d, w1d, b1d,
                        p['dock'], False)
    mwp_b, cols_b = _p4(enew_i, gsb, grb, rcol_i, m16_d, w1d, b1d,
                        p['dock'], False)

    out16 = _t4(nodes_d, (mwp_a, mwp_b), cols_a, cols_b, p)
    return out16[0, :7]


# speedup vs baseline: 5.6055x; 1.0188x over previous
import functools

import jax
import jax.numpy as jnp
from jax import lax
from jax.experimental import pallas as pl
from jax.experimental.pallas import tpu as pltpu
from jax.experimental.pallas import tpu_sc as plsc

C = 128
H = 8
DH = 16
N = 800
NWORK = 32
EB = 256
F32 = jnp.float32
BF16 = jnp.bfloat16


def _sel_dh_to_h():
    d = lax.broadcasted_iota(jnp.int32, (C, 16), 0)
    h = lax.broadcasted_iota(jnp.int32, (C, 16), 1)
    return jnp.where(d // DH == h, 1.0, 0.0).astype(F32)


def _sel_h_to_dh():
    h = lax.broadcasted_iota(jnp.int32, (16, C), 0)
    d = lax.broadcasted_iota(jnp.int32, (16, C), 1)
    return jnp.where((d // DH == h) & (h < H), 1.0, 0.0).astype(F32)


def _headmax_global(nodes, wq, wk):
    q = jnp.dot(nodes, wq[...], preferred_element_type=F32)
    k = jnp.dot(nodes, wk[...], preferred_element_type=F32)
    m16 = jnp.zeros((1, 16), F32)
    for h in range(H):
        qh = q[:, h * DH:(h + 1) * DH]
        kh = k[:, h * DH:(h + 1) * DH]
        at = lax.dot_general(kh, qh, (((1,), (1,)), ((), ())),
                             preferred_element_type=F32)
        mh = jnp.max(jnp.max(at, axis=1, keepdims=True), axis=0,
                     keepdims=True) * 0.25
        hh = lax.broadcasted_iota(jnp.int32, (1, 16), 1)
        oh = jnp.where(hh == h, 1.0, 0.0).astype(F32)
        m16 = m16 + mh * oh
    return m16


def _t1_body(n_rec, n_lig, wn, bn, wq, wk, we, be, w1e_s, b1_s, w1e_i, b1_i,
             nodes_o, m16_o, w1fs_o, b1fs_o, w1fi_o, b1fi_o):
    nr = jnp.dot(n_rec[...], wn[...], preferred_element_type=F32) + bn[...]
    nl = jnp.dot(n_lig[...], wn[...], preferred_element_type=F32) + bn[...]
    nodes = jnp.concatenate([nr, nl], axis=0)
    nodes_o[...] = nodes
    m16_o[...] = _headmax_global(nodes, wq, wk)
    w1fs_o[...] = jnp.dot(we[...], w1e_s[...], preferred_element_type=F32)
    b1fs_o[...] = jnp.dot(be[...], w1e_s[...], preferred_element_type=F32) + b1_s[...]
    w1fi_o[...] = jnp.dot(we[...], w1e_i[...], preferred_element_type=F32)
    b1fi_o[...] = jnp.dot(be[...], w1e_i[...], preferred_element_type=F32) + b1_i[...]


def _t1(n_rec, n_lig, p):
    args = (n_rec, n_lig, p['n_enc']['W'], p['n_enc']['b'][None, :],
            p['single']['Wq'], p['single']['Wk'],
            p['e_enc']['W'], p['e_enc']['b'][None, :],
            p['single']['edge']['W1'][:C], p['single']['edge']['b1'][None, :],
            p['inter']['edge']['W1'][:C], p['inter']['edge']['b1'][None, :])
    outs = [jax.ShapeDtypeStruct((N, C), F32), jax.ShapeDtypeStruct((1, 16), F32),
            jax.ShapeDtypeStruct((16, C), F32), jax.ShapeDtypeStruct((1, C), F32),
            jax.ShapeDtypeStruct((16, C), F32), jax.ShapeDtypeStruct((1, C), F32)]
    return pl.pallas_call(_t1_body, out_shape=outs)(*args)


def _nodeup_body(nodes, mw0, wna, wnb, bn1, wn2, bn2):
    mwp = mw0[...]
    aggu = mwp[:, :C]
    den16 = mwp[:, C:C + 16]
    r16 = _sel_h_to_dh()
    denr = jnp.dot(den16, r16, preferred_element_type=F32) + 1e-9
    agg = aggu / denr
    h1 = jnp.maximum(
        jnp.dot(nodes[...], wna[...], preferred_element_type=F32)
        + jnp.dot(agg, wnb[...], preferred_element_type=F32) + bn1[...], 0.0)
    return jnp.dot(h1, wn2[...], preferred_element_type=F32) + bn2[...]


def _node_mlp_args(nodes, mwp, pst):
    w1 = pst['node']['W1']
    return (nodes, mwp,
            w1[:C], w1[C:], pst['node']['b1'][None, :],
            pst['node']['W2'], pst['node']['b2'][None, :])


def _t2_body(nodes, mw0, wna, wnb, bn1, wn2, bn2,
             colsums, wg_n, wg_e, bg, wq, wk,
             ns_o, m16_o, gr_o, gl_o):
    ns = _nodeup_body(nodes, mw0, wna, wnb, bn1, wn2, bn2)
    ns_o[...] = ns
    m16_o[...] = _headmax_global(ns, wq, wk)
    nrm = jnp.mean(ns[0:400], axis=0, keepdims=True)
    nlm = jnp.mean(ns[400:800], axis=0, keepdims=True)
    erm = jnp.sum(colsums[0:200], axis=0) * (1.0 / 51200.0)
    elm = jnp.sum(colsums[200:400], axis=0) * (1.0 / 51200.0)
    gr_o[...] = (jnp.dot(nrm, wg_n[...], preferred_element_type=F32)
                 + jnp.dot(erm, wg_e[...], preferred_element_type=F32) + bg[...])
    gl_o[...] = (jnp.dot(nlm, wg_n[...], preferred_element_type=F32)
                 + jnp.dot(elm, wg_e[...], preferred_element_type=F32) + bg[...])


def _t2(nodes, mwp, colsums, p):
    pst = p['single']
    wg = pst['glob']['W']
    args = (*_node_mlp_args(nodes, mwp, pst), colsums,
            wg[:C], wg[C:], pst['glob']['b'][None, :],
            p['inter']['Wq'], p['inter']['Wk'])
    outs = [jax.ShapeDtypeStruct((N, C), F32), jax.ShapeDtypeStruct((1, 16), F32),
            jax.ShapeDtypeStruct((1, C), F32), jax.ShapeDtypeStruct((1, C), F32)]
    return pl.pallas_call(_t2_body, out_shape=outs)(*args)


def _t3_body(nodes, mw0, wna, wnb, bn1, wn2, bn2,
             colsums, wg_n, wg_e, bg, wq, wk,
             nodesd_o, m16_o, gi_o):
    ni = _nodeup_body(nodes, mw0, wna, wnb, bn1, wn2, bn2)
    nodesd = nodes[...] + ni
    nodesd_o[...] = nodesd
    m16_o[...] = _headmax_global(nodesd, wq, wk)
    nim = jnp.mean(ni, axis=0, keepdims=True)
    eim = jnp.sum(colsums[...], axis=0) * (1.0 / 51200.0)
    gi_o[...] = (jnp.dot(nim, wg_n[...], preferred_element_type=F32)
                 + jnp.dot(eim, wg_e[...], preferred_element_type=F32) + bg[...])


def _t3(nodes, mwp, colsums, p):
    pst = p['inter']
    wg = pst['glob']['W']
    args = (*_node_mlp_args(nodes, mwp, pst), colsums,
            wg[:C], wg[C:], pst['glob']['b'][None, :],
            p['dock']['Wq'], p['dock']['Wk'])
    outs = [jax.ShapeDtypeStruct((N, C), F32), jax.ShapeDtypeStruct((1, 16), F32),
            jax.ShapeDtypeStruct((1, C), F32)]
    return pl.pallas_call(_t3_body, out_shape=outs)(*args)


def _sigmoid(x):
    return 1.0 / (1.0 + jnp.exp(-x))


def _mlp_in(pm):
    return (pm['W1'], pm['b1'][None, :], pm['W2'], pm['b2'][None, :])


def _t4_body(nodes, mw0, mw1,
             wna, wnb, bn1, wn2, bn2, colsa, colsb, wg_n, wg_e, bg,
             rw1, rb1, rw2, rb2, tw1, tb1, tw2, tb2, cw1, cb1, cw2, cb2,
             wtr, btr, wro, bro, wco, bco, out_o):
    mwp = mw0[...] + mw1[...]
    aggu = mwp[:, :C]
    den16 = mwp[:, C:C + 16]
    r16 = _sel_h_to_dh()
    denr = jnp.dot(den16, r16, preferred_element_type=F32) + 1e-9
    agg = aggu / denr
    h1 = jnp.maximum(
        jnp.dot(nodes[...], wna[...], preferred_element_type=F32)
        + jnp.dot(agg, wnb[...], preferred_element_type=F32) + bn1[...], 0.0)
    nd = jnp.dot(h1, wn2[...], preferred_element_type=F32) + bn2[...]
    ndm = jnp.mean(nd, axis=0, keepdims=True)
    edm = ((jnp.sum(colsa[...], axis=0) + jnp.sum(colsb[...], axis=0))
           * (1.0 / 153600.0))
    gd = (jnp.dot(ndm, wg_n[...], preferred_element_type=F32)
          + jnp.dot(edm, wg_e[...], preferred_element_type=F32) + bg[...])

    def mlp(x, w1, b1, w2, b2):
        hh = jnp.maximum(jnp.dot(x, w1[...], preferred_element_type=F32) + b1[...], 0.0)
        return jnp.dot(hh, w2[...], preferred_element_type=F32) + b2[...]

    out_r = mlp(gd, rw1, rb1, rw2, rb2)
    out_t = mlp(gd + out_r, tw1, tb1, tw2, tb2)
    out_c = mlp(gd, cw1, cb1, cw2, cb2)
    t3 = _sigmoid(jnp.dot(out_t, wtr[...], preferred_element_type=F32) + btr[...]) * 2.0 - 1.0
    r3 = _sigmoid(jnp.dot(out_r, wro[...], preferred_element_type=F32) + bro[...]) * 0.2 - 0.1
    c1 = _sigmoid(jnp.dot(out_c, wco[...], preferred_element_type=F32) + bco[...])
    i = lax.broadcasted_iota(jnp.int32, (16, 16), 0)
    j = lax.broadcasted_iota(jnp.int32, (16, 16), 1)
    p_r = jnp.where((i == j) & (i < 3), 1.0, 0.0).astype(F32)
    p_t = jnp.where((j == i + 3) & (i < 3), 1.0, 0.0).astype(F32)
    p_c = jnp.where((i == 0) & (j == 6), 1.0, 0.0).astype(F32)
    out_o[...] = (jnp.dot(r3, p_r, preferred_element_type=F32)
                  + jnp.dot(t3, p_t, preferred_element_type=F32)
                  + jnp.dot(c1, p_c, preferred_element_type=F32))


def _pad16(w):
    return jnp.pad(w, ((0, 0), (0, 16 - w.shape[1])))


def _t4(nodes, mwps, colsa, colsb, p):
    pst = p['dock']
    w1 = pst['node']['W1']
    wg = pst['glob']['W']
    args = (nodes, mwps[0], mwps[1],
            w1[:C], w1[C:], pst['node']['b1'][None, :],
            pst['node']['W2'], pst['node']['b2'][None, :],
            colsa, colsb, wg[:C], wg[C:], pst['glob']['b'][None, :],
            *_mlp_in(p['out_r']), *_mlp_in(p['out_t']), *_mlp_in(p['out_c']),
            _pad16(p['traslator']['W']), _pad16(p['traslator']['b'][None, :]),
            _pad16(p['rotator']['W']), _pad16(p['rotator']['b'][None, :]),
            _pad16(p['confidence']['W']), _pad16(p['confidence']['b'][None, :]))
    out = pl.pallas_call(_t4_body, out_shape=[jax.ShapeDtypeStruct((1, 16), F32)])(*args)
    return out[0]


def _p4_body_mk(din, write_enew):
    def body(*refs):
        if write_enew:
            (eraw, gs, gr, rcol, m16, w1f, b1f, wsq, wrk, w2, b2,
             enew_o, agg_o, cols_o) = refs
        else:
            (eraw, gs, gr, rcol, m16, w1f, b1f, wsq, wrk, w2, b2,
             agg_o, cols_o) = refs
        x = jnp.dot(eraw[...], w1f[...], preferred_element_type=F32) + b1f[...]
        gsv = gs[...].astype(BF16)
        grv = gr[...].astype(BF16)
        ts = jnp.dot(gsv, wsq[...], preferred_element_type=F32)
        tr = jnp.dot(grv, wrk[...], preferred_element_type=F32)
        h1 = jnp.maximum(x + ts[:, :C] + tr[:, :C], 0.0)
        enew = jnp.dot(h1.astype(BF16), w2[...],
                       preferred_element_type=F32) + b2[...]
        l16 = jnp.dot(ts[:, C:] * tr[:, C:], _sel_dh_to_h(),
                      preferred_element_type=F32) * 0.25
        w16 = jnp.exp(l16 - m16[...])
        msgu = enew * jnp.dot(w16, _sel_h_to_dh(), preferred_element_type=F32)
        nn = lax.broadcasted_iota(jnp.int32, (EB, N), 1)
        onehot = jnp.where(rcol[...] == nn, 1.0, 0.0).astype(BF16)
        mw = jnp.concatenate([msgu, w16[:, :16]], axis=1).astype(BF16)
        contrib = lax.dot_general(onehot, mw, (((0,), (0,)), ((), ())),
                                  preferred_element_type=F32)

        @pl.when(pl.program_id(0) == 0)
        def _zero():
            agg_o[...] = jnp.zeros((N, 144), F32)

        agg_o[...] += contrib
        if write_enew:
            enew_o[...] = enew.astype(BF16)
        cols_o[...] = jnp.sum(enew, axis=0, keepdims=True)[None, :, :]
    return body


def _p4(eraw, gs, gr, rcol, m16, w1f, b1f, pst, write_enew):
    e = eraw.shape[0]
    din = eraw.shape[1]
    nb = e // EB
    w1 = pst['edge']['W1']
    wsq = jnp.concatenate([w1[C:2 * C], pst['Wq']], axis=1).astype(BF16)
    wrk = jnp.concatenate([w1[2 * C:], pst['Wk']], axis=1).astype(BF16)
    w2 = pst['edge']['W2'].astype(BF16)
    if din == C:
        eraw = eraw.astype(BF16)
        w1f = w1f.astype(BF16)
    full = lambda s: pl.BlockSpec(s, lambda i: tuple(0 for _ in s))
    in_specs = [pl.BlockSpec((EB, din), lambda i: (i, 0)),
                pl.BlockSpec((EB, C), lambda i: (i, 0)),
                pl.BlockSpec((EB, C), lambda i: (i, 0)),
                pl.BlockSpec((EB, 1), lambda i: (i, 0)),
                full((1, 16)),
                full((din, C)), full((1, C)), full((C, 256)), full((C, 256)),
                full((C, C)), full((1, C))]
    out_shapes = []
    out_specs = []
    if write_enew:
        out_shapes.append(jax.ShapeDtypeStruct((e, C), BF16))
        out_specs.append(pl.BlockSpec((EB, C), lambda i: (i, 0)))
    out_shapes += [jax.ShapeDtypeStruct((N, 144), F32),
                   jax.ShapeDtypeStruct((nb, 1, C), F32)]
    out_specs += [pl.BlockSpec((N, 144), lambda i: (0, 0)),
                  pl.BlockSpec((1, 1, C), lambda i: (i, 0, 0))]
    args = (eraw, gs, gr, rcol, m16, w1f, b1f, wsq, wrk,
            w2, pst['edge']['b2'][None, :])
    return pl.pallas_call(
        _p4_body_mk(din, write_enew), grid=(nb,),
        in_specs=in_specs, out_specs=out_specs, out_shape=out_shapes)(*args)


def _mesh():
    return plsc.VectorSubcoreMesh(core_axis_name="c", subcore_axis_name="s")


def _sc_gather(table, sidx, ridx):
    nch = sidx.shape[1]
    ch = sidx.shape[2]
    e = NWORK * nch * ch
    per = nch * ch
    dt = F32

    @functools.partial(
        pl.kernel, mesh=_mesh(),
        out_type=[jax.ShapeDtypeStruct((e, C), dt),
                  jax.ShapeDtypeStruct((e, C), dt)],
        scratch_types=[pltpu.VMEM((nch, ch), jnp.int32),
                       pltpu.VMEM((nch, ch), jnp.int32),
                       pltpu.VMEM((ch, C), dt),
                       pltpu.VMEM((ch, C), dt),
                       pltpu.VMEM((ch, C), dt),
                       pltpu.VMEM((ch, C), dt),
                       pltpu.SemaphoreType.DMA,
                       pltpu.SemaphoreType.DMA,
                       pltpu.SemaphoreType.DMA,
                       pltpu.SemaphoreType.DMA],
    )
    def k(table_h, sidx_h, ridx_h, gs_h, gr_h,
          sv, rv, bs0, br0, bs1, br1, ss0, sr0, ss1, sr1):
        wid = lax.axis_index("s") * 2 + lax.axis_index("c")
        base = wid * per
        pltpu.sync_copy(sidx_h.at[wid], sv)
        pltpu.sync_copy(ridx_h.at[wid], rv)

        def issue(kk, bs, br, sems, semr):
            pltpu.async_copy(table_h.at[sv.at[kk]], bs, sems)
            pltpu.async_copy(table_h.at[rv.at[kk]], br, semr)

        def drain(bs, br, sems, semr):
            pltpu.make_async_copy(table_h.at[pl.ds(0, ch)], bs, sems).wait()
            pltpu.make_async_copy(table_h.at[pl.ds(0, ch)], br, semr).wait()

        def flush(kk, bs, br):
            pltpu.sync_copy(bs, gs_h.at[pl.ds(base + kk * ch, ch)])
            pltpu.sync_copy(br, gr_h.at[pl.ds(base + kk * ch, ch)])

        issue(0, bs0, br0, ss0, sr0)

        def pair(i, carry):
            k0 = i * 2
            issue(k0 + 1, bs1, br1, ss1, sr1)
            drain(bs0, br0, ss0, sr0)
            flush(k0, bs0, br0)

            @pl.when(k0 + 2 < nch)
            def _next():
                issue(k0 + 2, bs0, br0, ss0, sr0)

            drain(bs1, br1, ss1, sr1)
            flush(k0 + 1, bs1, br1)
            return carry

        lax.fori_loop(0, nch // 2, pair, 0)

    return k(table, sidx, ridx)


def _tiles(idx, ch):
    return idx.astype(jnp.int32).reshape(NWORK, -1, ch)


def kernel(e_rec, s_rec, r_rec, n_rec, e_lig, s_lig, r_lig, n_lig,
           e_int, s_int, r_int, params):
    p = params

    s_s = jnp.concatenate([s_rec, s_lig + 400])
    r_s = jnp.concatenate([r_rec, r_lig + 400])
    sidx_s, ridx_s = _tiles(s_s, 64), _tiles(r_s, 64)
    sidx_i, ridx_i = _tiles(s_int, 32), _tiles(r_int, 32)
    rcol_s = r_s.astype(jnp.int32)[:, None]
    rcol_i = r_int.astype(jnp.int32)[:, None]

    nodes_s, m16_s, w1fs, b1fs, w1fi, b1fi = _t1(n_rec, n_lig, p)
    gs, gr = _sc_gather(nodes_s, sidx_s, ridx_s)
    eraw_s = jnp.concatenate([e_rec, e_lig], axis=0)
    enew_s, mwp, cols_s = _p4(eraw_s, gs, gr, rcol_s, m16_s, w1fs, b1fs,
                              p['single'], True)
    ns, m16_i, gr_g, gl_g = _t2(nodes_s, mwp, cols_s, p)

    gs, gr = _sc_gather(ns, sidx_i, ridx_i)
    enew_i, mwp, cols_i = _p4(e_int, gs, gr, rcol_i, m16_i, w1fi, b1fi,
                              p['inter'], True)
    nodes_d, m16_d, gi_g = _t3(ns, mwp, cols_i, p)

    w1d = p['dock']['edge']['W1'][:C]
    b1d = p['dock']['edge']['b1'][None, :]
    gsa, gra = _sc_gather(nodes_d, sidx_s, ridx_s)
    gsb, grb = _sc_gather(nodes_d, sidx_i, ridx_i)
    mwp_a, cols_a = _p4(enew_s, gsa, gra, rcol_s, m16_d, w1d, b1d,
                        p['dock'], False)
    mwp_b, cols_b = _p4(enew_i, gsb, grb, rcol_i, m16_d, w1d, b1d,
                        p['dock'], False)

    out16 = _t4(nodes_d, (mwp_a, mwp_b), cols_a, cols_b, p)
    return out16[0, :7]
